# Initial kernel scaffold; baseline (speedup 1.0000x reference)
import jax
import jax.numpy as jnp
from jax.experimental import pallas as pl


def kernel(x, w_u):
    raise NotImplementedError("write your pallas kernel here")


# speedup vs baseline: 2.1830x; 2.1830x over previous
import jax
import jax.numpy as jnp
from jax.experimental import pallas as pl
from jax.experimental.pallas import tpu as pltpu


def _unembed_tile(x_ref, w_ref, o_ref):
    o_ref[...] = jnp.dot(
        x_ref[...], w_ref[...], preferred_element_type=jnp.float32
    )


def kernel(x, w_u):
    b, p, d_emb = x.shape
    d_emb_w, d_vocab = w_u.shape
    assert d_emb == d_emb_w

    rows = b * p
    x2d = x.reshape(rows, d_emb)

    tm = min(2048, rows)
    tn = 512

    grid = (pl.cdiv(rows, tm), pl.cdiv(d_vocab, tn))

    out2d = pl.pallas_call(
        _unembed_tile,
        grid=grid,
        in_specs=[
            pl.BlockSpec((tm, d_emb), lambda i, j: (i, 0)),
            pl.BlockSpec((d_emb, tn), lambda i, j: (0, j)),
        ],
        out_specs=pl.BlockSpec((tm, tn), lambda i, j: (i, j)),
        out_shape=jax.ShapeDtypeStruct((rows, d_vocab), jnp.float32),
        compiler_params=pltpu.CompilerParams(
            dimension_semantics=("parallel", "parallel"),
            vmem_limit_bytes=56 * 1024 * 1024,
        ),
    )(x2d, w_u)

    return out2d.reshape(b, p, d_vocab)
